# baseline (speedup 1.0000x reference)
import functools

import jax
import jax.numpy as jnp
from jax import lax
from jax.experimental import pallas as pl
from jax.experimental.pallas import tpu as pltpu
from jax.experimental.pallas import tpu_sc as plsc

N = 10000
E = 320000
H = 128

NC = 2
NS = 16
L = 16
NW = NC * NS

NP = 10240
EPT = E // NW
CH = 2000
C = 80
C2 = 40
CB = 400
HB = CB // C2
NPT = NP // NS
SROWS = 16

NBLK = 2048
EBLK = 8192


def _node_proj_kernel(nf_ref, WnT_ref, bn_ref, Wab_ref, cab_ref,
                      x_ref, a_ref, b_ref):
    x = (
        jnp.dot(nf_ref[...], WnT_ref[...], preferred_element_type=jnp.float32)
        + bn_ref[...]
    )
    x_ref[...] = x
    ab = jnp.dot(x, Wab_ref[...], preferred_element_type=jnp.float32)
    a_ref[...] = ab[:, 0]
    b_ref[...] = ab[:, 1]


def _node_proj(node_feat, WnT, bn, Wab, cab):
    return pl.pallas_call(
        _node_proj_kernel,
        grid=(NP // NBLK,),
        in_specs=[
            pl.BlockSpec((NBLK, H), lambda i: (i, 0)),
            pl.BlockSpec((H, H), lambda i: (0, 0)),
            pl.BlockSpec((1, H), lambda i: (0, 0)),
            pl.BlockSpec((H, 2), lambda i: (0, 0)),
            pl.BlockSpec((1, 2), lambda i: (0, 0)),
        ],
        out_specs=[
            pl.BlockSpec((NBLK, H), lambda i: (i, 0)),
            pl.BlockSpec((NBLK,), lambda i: (i,)),
            pl.BlockSpec((NBLK,), lambda i: (i,)),
        ],
        out_shape=[
            jax.ShapeDtypeStruct((N, H), jnp.float32),
            jax.ShapeDtypeStruct((NP,), jnp.float32),
            jax.ShapeDtypeStruct((NP,), jnp.float32),
        ],
    )(node_feat, WnT, bn, Wab, cab)


def _ce_proj_kernel(ef_ref, gv_ref, WeAB_ref, Wg2_ref, cg_ref,
                    ce1_ref, ce2_ref):
    ce_e = jnp.dot(ef_ref[...], WeAB_ref[...], preferred_element_type=jnp.float32)
    ce_g = jnp.dot(gv_ref[...], Wg2_ref[...], preferred_element_type=jnp.float32)
    ce = ce_e + ce_g + cg_ref[...]
    ce1_ref[...] = ce[:, 0]
    ce2_ref[...] = ce[:, 1]


def _ce_proj(edge_feat, guild_vec, WeAB, Wg2, cg):
    return pl.pallas_call(
        _ce_proj_kernel,
        grid=(pl.cdiv(E, EBLK),),
        in_specs=[
            pl.BlockSpec((EBLK, H), lambda i: (i, 0)),
            pl.BlockSpec((EBLK, H), lambda i: (i, 0)),
            pl.BlockSpec((H, 2), lambda i: (0, 0)),
            pl.BlockSpec((H, 2), lambda i: (0, 0)),
            pl.BlockSpec((1, 2), lambda i: (0, 0)),
        ],
        out_specs=[
            pl.BlockSpec((EBLK,), lambda i: (i,)),
            pl.BlockSpec((EBLK,), lambda i: (i,)),
        ],
        out_shape=[
            jax.ShapeDtypeStruct((E,), jnp.float32),
            jax.ShapeDtypeStruct((E,), jnp.float32),
        ],
    )(edge_feat, guild_vec, WeAB, Wg2, cg)


def _edge_final_kernel(ef_ref, WeT_ref, be_ref, d_ref, out_ref):
    out_ref[...] = (
        jnp.dot(ef_ref[...], WeT_ref[...], preferred_element_type=jnp.float32)
        + be_ref[...]
        + d_ref[...]
    )


def _edge_final(edge_feat, WeT, be, d):
    return pl.pallas_call(
        _edge_final_kernel,
        grid=(pl.cdiv(E, EBLK),),
        in_specs=[
            pl.BlockSpec((EBLK, H), lambda i: (i, 0)),
            pl.BlockSpec((H, H), lambda i: (0, 0)),
            pl.BlockSpec((1, H), lambda i: (0, 0)),
            pl.BlockSpec((EBLK, H), lambda i: (i, 0)),
        ],
        out_specs=pl.BlockSpec((EBLK, H), lambda i: (i, 0)),
        out_shape=jax.ShapeDtypeStruct((E, H), jnp.float32),
    )(edge_feat, WeT, be, d)


def _final_kernel(hp_ref, x_ref, h_ref):
    h_ref[...] = hp_ref[0, :, :] + hp_ref[1, :, :] + x_ref[...]


def _final_add(hpart, x):
    return pl.pallas_call(
        _final_kernel,
        grid=(5,),
        in_specs=[
            pl.BlockSpec((2, 2000, H), lambda i: (0, i, 0)),
            pl.BlockSpec((2000, H), lambda i: (i, 0)),
        ],
        out_specs=pl.BlockSpec((2000, H), lambda i: (i, 0)),
        out_shape=jax.ShapeDtypeStruct((N, H), jnp.float32),
    )(hpart, x)


def _lrelu(v):
    return jnp.where(v >= 0.0, v, 0.01 * v)


def _sc_mesh():
    return plsc.VectorSubcoreMesh(core_axis_name="c", subcore_axis_name="s")


def _sc_scalar_pass(a_pad, b_pad, src, dst, ce1, ce2):
    @functools.partial(
        pl.kernel,
        out_type=[
            jax.ShapeDtypeStruct((E,), jnp.float32),
            jax.ShapeDtypeStruct((E,), jnp.float32),
            jax.ShapeDtypeStruct((NC, L), jnp.float32),
            jax.ShapeDtypeStruct((NW, NP), jnp.float32),
        ],
        mesh=_sc_mesh(),
        compiler_params=pltpu.CompilerParams(needs_layout_passes=False),
        scratch_types=[
            pltpu.VMEM((NP,), jnp.float32),
            pltpu.VMEM((NP,), jnp.float32),
            pltpu.VMEM((EPT,), jnp.int32),
            pltpu.VMEM((EPT,), jnp.float32),
            pltpu.VMEM((NP,), jnp.float32),
            pltpu.VMEM((CH,), jnp.int32),
            pltpu.VMEM((CH,), jnp.float32),
            pltpu.VMEM((CH,), jnp.float32),
            pltpu.VMEM((CH,), jnp.float32),
            pltpu.VMEM((CH,), jnp.float32),
            pltpu.VMEM((L,), jnp.float32),
            pltpu.VMEM((NS, L), jnp.float32),
            pltpu.VMEM_SHARED((NS, L), jnp.float32),
        ],
    )
    def body(a_hbm, b_hbm, src_hbm, dst_hbm, ce1_hbm, ce2_hbm,
             alpha_out, e_out, mc_out, pssum_out,
             a_tab, b_tab, dst_tab, score_tab, ssum_tab,
             src_c, ce1_c, ce2_c, alpha_c, e_c, rmax, maxloc, max_sh):
        c = lax.axis_index("c")
        s = lax.axis_index("s")
        w = s * NC + c
        ebase = w * EPT

        pltpu.sync_copy(a_hbm, a_tab)
        pltpu.sync_copy(b_hbm, b_tab)
        rmax[...] = jnp.full((L,), -3e38, jnp.float32)

        def chunk_body(i, carry):
            off = ebase + i * CH
            pltpu.sync_copy(src_hbm.at[pl.ds(off, CH)], src_c)
            pltpu.sync_copy(dst_hbm.at[pl.ds(off, CH)], dst_tab.at[pl.ds(i * CH, CH)])
            pltpu.sync_copy(ce1_hbm.at[pl.ds(off, CH)], ce1_c)
            pltpu.sync_copy(ce2_hbm.at[pl.ds(off, CH)], ce2_c)

            def grp(g, carry2):
                sl = pl.ds(g * L, L)
                si = src_c[sl]
                di = dst_tab[pl.ds(i * CH + g * L, L)]
                a_s = plsc.load_gather(a_tab, [si])
                a_d = plsc.load_gather(a_tab, [di])
                b_s = plsc.load_gather(b_tab, [si])
                b_d = plsc.load_gather(b_tab, [di])
                c1 = ce1_c[sl]
                c2 = ce2_c[sl]
                z1 = _lrelu(a_s + c1)
                z2 = _lrelu(a_d + c1)
                dz = z1 - z2
                p = jnp.exp(-jnp.abs(dz))
                q = 1.0 / (1.0 + p)
                al = jnp.where(dz >= 0.0, q, p * q)
                sc_ = _lrelu((1.0 + al) * b_s + (2.0 - al) * b_d + c2)
                alpha_c[sl] = al
                score_tab[pl.ds(i * CH + g * L, L)] = sc_
                rmax[...] = jnp.maximum(rmax[...], sc_)
                return carry2

            lax.fori_loop(0, CH // L, grp, 0)
            pltpu.sync_copy(alpha_c, alpha_out.at[pl.ds(off, CH)])
            return carry

        lax.fori_loop(0, EPT // CH, chunk_body, 0)

        pltpu.sync_copy(rmax, max_sh.at[s])
        plsc.subcore_barrier()
        pltpu.sync_copy(max_sh, maxloc)
        mv = maxloc[0, :]
        for k in range(1, NS):
            mv = jnp.maximum(mv, maxloc[k, :])
        mc = jnp.max(mv)
        mcv = jnp.broadcast_to(mc, (L,))

        @pl.when(s == 0)
        def _():
            rmax[...] = mcv
            pltpu.sync_copy(rmax, mc_out.at[c])

        def zr(g, carry):
            ssum_tab[pl.ds(g * L, L)] = jnp.zeros((L,), jnp.float32)
            return carry

        lax.fori_loop(0, NP // L, zr, 0)

        def chunk2(i, carry):
            def grp2(g, carry2):
                sl_t = pl.ds(i * CH + g * L, L)
                ev = jnp.exp(score_tab[sl_t] - mcv)
                e_c[pl.ds(g * L, L)] = ev
                di = dst_tab[sl_t]
                plsc.addupdate_scatter(ssum_tab, [di], ev)
                return carry2

            lax.fori_loop(0, CH // L, grp2, 0)
            pltpu.sync_copy(e_c, e_out.at[pl.ds(ebase + i * CH, CH)])
            return carry

        lax.fori_loop(0, EPT // CH, chunk2, 0)
        pltpu.sync_copy(ssum_tab, pssum_out.at[w])

    return body(a_pad, b_pad, src, dst, ce1, ce2)


def _sc_row_pass(x, src, dst, alpha0, e, mc, pssum):
    @functools.partial(
        pl.kernel,
        out_type=[
            jax.ShapeDtypeStruct((E, H), jnp.float32),
            jax.ShapeDtypeStruct((NC, NP, H), jnp.float32),
        ],
        mesh=_sc_mesh(),
        compiler_params=pltpu.CompilerParams(needs_layout_passes=False),
        scratch_types=[
            pltpu.VMEM((NP,), jnp.float32),
            pltpu.VMEM((NPT,), jnp.float32),
            pltpu.VMEM((NPT,), jnp.float32),
            pltpu.VMEM((NPT,), jnp.float32),
            pltpu.VMEM((NC, L), jnp.float32),
            pltpu.VMEM((CB,), jnp.int32),
            pltpu.VMEM((CB,), jnp.int32),
            pltpu.VMEM((CB,), jnp.float32),
            pltpu.VMEM((CB,), jnp.float32),
            pltpu.VMEM((2, C2), jnp.int32),
            pltpu.VMEM((C, H), jnp.float32),
            pltpu.VMEM((C, H), jnp.float32),
            pltpu.VMEM((C, H), jnp.float32),
            pltpu.VMEM((SROWS, H), jnp.float32),
            pltpu.VMEM_SHARED((NP,), jnp.float32),
            pltpu.VMEM_SHARED((NP, H), jnp.float32),
            pltpu.SemaphoreType.DMA((2,)),
            pltpu.SemaphoreType.DMA((2,)),
            pltpu.SemaphoreType.DMA((2,)),
            pltpu.SemaphoreType.DMA((2,)),
            pltpu.SemaphoreType.DMA((2,)),
        ],
    )
    def body(x_hbm, src_hbm, dst_hbm, al_hbm, e_hbm, mc_hbm, pssum_hbm,
             d_out, hpart_out,
             rinv_tab, row_buf, acc0_b, acc1_b, mcl, src_c, dst_c, al_c,
             bel_c, dst_ch, xs_b, xd_b, d_b, stage, rinv_sh, h_sh,
             lsem_xs, lsem_xd, lsem_dc, wsem_d, wsem_sc):
        c = lax.axis_index("c")
        s = lax.axis_index("s")
        w = s * NC + c
        ebase = w * EPT
        nb = s * NPT

        pltpu.sync_copy(mc_hbm, mcl)
        m0 = mcl[0, :]
        m1 = mcl[1, :]
        mg = jnp.maximum(m0, m1)
        sc0 = jnp.exp(m0 - mg)
        sc1 = jnp.exp(m1 - mg)
        cv = jnp.broadcast_to(c, (L,))
        myscale = jnp.where(cv == 0, sc0, sc1)

        def zacc(g, carry):
            sl = pl.ds(g * L, L)
            acc0_b[sl] = jnp.zeros((L,), jnp.float32)
            acc1_b[sl] = jnp.zeros((L,), jnp.float32)
            return carry

        lax.fori_loop(0, NPT // L, zacc, 0)
        for w2 in range(NW):
            pltpu.sync_copy(pssum_hbm.at[w2, pl.ds(nb, NPT)], row_buf)

            def accg(g, carry, _w2=w2):
                sl = pl.ds(g * L, L)
                if _w2 % NC == 0:
                    acc0_b[sl] = acc0_b[sl] + row_buf[sl]
                else:
                    acc1_b[sl] = acc1_b[sl] + row_buf[sl]
                return carry

            lax.fori_loop(0, NPT // L, accg, 0)

        def cg(g, carry):
            sl = pl.ds(g * L, L)
            tot = acc0_b[sl] * sc0 + acc1_b[sl] * sc1
            row_buf[sl] = 1.0 / jnp.maximum(tot, 1e-16)
            return carry

        lax.fori_loop(0, NPT // L, cg, 0)
        pltpu.sync_copy(row_buf, rinv_sh.at[pl.ds(nb, NPT)])

        def zs(r, carry):
            for j in range(H // L):
                stage[r, pl.ds(j * L, L)] = jnp.zeros((L,), jnp.float32)
            return carry

        lax.fori_loop(0, SROWS, zs, 0)
        for k in range(NPT // SROWS):
            pltpu.sync_copy(stage, h_sh.at[pl.ds(nb + k * SROWS, SROWS), :])

        plsc.subcore_barrier()
        pltpu.sync_copy(rinv_sh, rinv_tab)

        def _load_descs(boff, i):
            p = i % 2
            rows = pl.ds(p * C2, C2)
            off = boff + i * C2
            return [
                pltpu.make_async_copy(
                    x_hbm.at[src_c.at[pl.ds(i * C2, C2)]],
                    xs_b.at[rows, :], lsem_xs.at[p]),
                pltpu.make_async_copy(
                    x_hbm.at[dst_c.at[pl.ds(i * C2, C2)]],
                    xd_b.at[rows, :], lsem_xd.at[p]),
                pltpu.make_async_copy(
                    dst_hbm.at[pl.ds(off, C2)], dst_ch.at[p], lsem_dc.at[p]),
            ]

        def _start_writes(boff, i):
            p = i % 2
            rows = pl.ds(p * C2, C2)
            off = boff + i * C2
            pltpu.make_async_copy(
                d_b.at[rows, :], d_out.at[pl.ds(off, C2), :],
                wsem_d.at[p]).start()
            pltpu.async_copy(
                xs_b.at[rows, :], h_sh.at[dst_ch.at[p]], wsem_sc.at[p],
                add=True)

        def _wait_writes(boff, i):
            p = i % 2
            rows = pl.ds(p * C2, C2)
            off = boff + i * C2
            pltpu.make_async_copy(
                d_b.at[rows, :], d_out.at[pl.ds(off, C2), :],
                wsem_d.at[p]).wait()
            pltpu.make_async_copy(
                xs_b.at[rows, :], h_sh.at[dst_ch.at[p]],
                wsem_sc.at[p]).wait()

        def bat(ib, carry):
            boff = ebase + ib * CB
            pltpu.sync_copy(src_hbm.at[pl.ds(boff, CB)], src_c)
            pltpu.sync_copy(dst_hbm.at[pl.ds(boff, CB)], dst_c)
            pltpu.sync_copy(al_hbm.at[pl.ds(boff, CB)], al_c)
            pltpu.sync_copy(e_hbm.at[pl.ds(boff, CB)], bel_c)

            def pg(g, carry2):
                sl = pl.ds(g * L, L)
                di = dst_c[sl]
                rv = plsc.load_gather(rinv_tab, [di])
                bel_c[sl] = bel_c[sl] * myscale * rv
                return carry2

            lax.fori_loop(0, CB // L, pg, 0)

            for d in _load_descs(boff, 0):
                d.start()

            def half(i, carry2):
                p = i % 2
                for d in _load_descs(boff, i):
                    d.wait()

                @pl.when(i + 1 < HB)
                def _():
                    @pl.when(i >= 1)
                    def _():
                        _wait_writes(boff, i - 1)

                    for d in _load_descs(boff, i + 1):
                        d.start()

                def rw(r, carry3):
                    rb = p * C2 + r
                    ridx = jnp.broadcast_to(i * C2 + r, (L,)).astype(jnp.int32)
                    alv = plsc.load_gather(al_c, [ridx])
                    blv = plsc.load_gather(bel_c, [ridx])
                    for j in range(H // L):
                        slj = pl.ds(j * L, L)
                        xsv = xs_b[rb, slj]
                        xdv = xd_b[rb, slj]
                        d_b[rb, slj] = alv * (xsv - xdv) + xdv
                        xs_b[rb, slj] = blv * xsv
                    return carry3

                lax.fori_loop(0, C2, rw, 0)
                _start_writes(boff, i)
                return carry2

            lax.fori_loop(0, HB, half, 0)
            _wait_writes(boff, HB - 2)
            _wait_writes(boff, HB - 1)
            return carry

        lax.fori_loop(0, EPT // CB, bat, 0)
        plsc.subcore_barrier()

        for k in range(NPT // SROWS):
            rows = pl.ds(nb + k * SROWS, SROWS)
            pltpu.sync_copy(h_sh.at[rows, :], stage)
            pltpu.sync_copy(stage, hpart_out.at[c, rows, :])

    return body(x, src, dst, alpha0, e, mc, pssum)


def kernel(node_feat, edge_feat, guild_vec, edge_index, Wn, bn, We, be, Wg, bg, Wna, Wea):
    src = edge_index[0].astype(jnp.int32)
    dst = edge_index[1].astype(jnp.int32)
    wea = Wea[0]
    wna = Wna[0]

    Wab = jnp.stack([wea, wna], axis=1)
    WeAB = We.T @ Wab
    Wg2 = Wg.T @ Wab
    cg = (((be + bg) @ Wab))[None, :]
    cab = jnp.zeros((1, 2), jnp.float32)

    x, a_pad, b_pad = _node_proj(node_feat, Wn.T, bn[None, :], Wab, cab)
    ce1, ce2 = _ce_proj(edge_feat, guild_vec, WeAB, Wg2, cg)

    alpha0, e, mc, pssum = _sc_scalar_pass(a_pad, b_pad, src, dst, ce1, ce2)
    d, hpart = _sc_row_pass(x, src, dst, alpha0, e, mc, pssum)
    ef_new = _edge_final(edge_feat, We.T, be[None, :], d)
    h = _final_add(hpart, x)
    return h, ef_new


# speedup vs baseline: 1.5467x; 1.5467x over previous
import functools

import jax
import jax.numpy as jnp
from jax import lax
from jax.experimental import pallas as pl
from jax.experimental.pallas import tpu as pltpu
from jax.experimental.pallas import tpu_sc as plsc

N = 10000
E = 320000
H = 128

NC = 2
NS = 16
L = 16
NW = NC * NS

NP = 10240
EPT = E // NW
CH = 2000
C = 80
C2 = 40
CB = 400
HB = CB // C2
NPT = NP // NS
SROWS = 16

NBLK = 2048
EBLK = 8192


def _node_proj_kernel(nf_ref, WnT_ref, bn_ref, Wab_ref, cab_ref,
                      x_ref, a_ref, b_ref):
    x = (
        jnp.dot(nf_ref[...], WnT_ref[...], preferred_element_type=jnp.float32)
        + bn_ref[...]
    )
    x_ref[...] = x
    ab = jnp.dot(x, Wab_ref[...], preferred_element_type=jnp.float32)
    a_ref[...] = ab[:, 0]
    b_ref[...] = ab[:, 1]


def _node_proj(node_feat, WnT, bn, Wab, cab):
    return pl.pallas_call(
        _node_proj_kernel,
        grid=(NP // NBLK,),
        in_specs=[
            pl.BlockSpec((NBLK, H), lambda i: (i, 0)),
            pl.BlockSpec((H, H), lambda i: (0, 0)),
            pl.BlockSpec((1, H), lambda i: (0, 0)),
            pl.BlockSpec((H, 2), lambda i: (0, 0)),
            pl.BlockSpec((1, 2), lambda i: (0, 0)),
        ],
        out_specs=[
            pl.BlockSpec((NBLK, H), lambda i: (i, 0)),
            pl.BlockSpec((NBLK,), lambda i: (i,)),
            pl.BlockSpec((NBLK,), lambda i: (i,)),
        ],
        out_shape=[
            jax.ShapeDtypeStruct((N, H), jnp.float32),
            jax.ShapeDtypeStruct((NP,), jnp.float32),
            jax.ShapeDtypeStruct((NP,), jnp.float32),
        ],
    )(node_feat, WnT, bn, Wab, cab)


def _ce_proj_kernel(ef_ref, gv_ref, WeAB_ref, Wg2_ref, cg_ref,
                    ce1_ref, ce2_ref):
    ce_e = jnp.dot(ef_ref[...], WeAB_ref[...], preferred_element_type=jnp.float32)
    ce_g = jnp.dot(gv_ref[...], Wg2_ref[...], preferred_element_type=jnp.float32)
    ce = ce_e + ce_g + cg_ref[...]
    ce1_ref[...] = ce[:, 0]
    ce2_ref[...] = ce[:, 1]


def _ce_proj(edge_feat, guild_vec, WeAB, Wg2, cg):
    return pl.pallas_call(
        _ce_proj_kernel,
        grid=(pl.cdiv(E, EBLK),),
        in_specs=[
            pl.BlockSpec((EBLK, H), lambda i: (i, 0)),
            pl.BlockSpec((EBLK, H), lambda i: (i, 0)),
            pl.BlockSpec((H, 2), lambda i: (0, 0)),
            pl.BlockSpec((H, 2), lambda i: (0, 0)),
            pl.BlockSpec((1, 2), lambda i: (0, 0)),
        ],
        out_specs=[
            pl.BlockSpec((EBLK,), lambda i: (i,)),
            pl.BlockSpec((EBLK,), lambda i: (i,)),
        ],
        out_shape=[
            jax.ShapeDtypeStruct((E,), jnp.float32),
            jax.ShapeDtypeStruct((E,), jnp.float32),
        ],
    )(edge_feat, guild_vec, WeAB, Wg2, cg)


def _edge_final_kernel(ef_ref, WeT_ref, be_ref, d_ref, out_ref):
    out_ref[...] = (
        jnp.dot(ef_ref[...], WeT_ref[...], preferred_element_type=jnp.float32)
        + be_ref[...]
        + d_ref[...]
    )


def _edge_final(edge_feat, WeT, be, d):
    return pl.pallas_call(
        _edge_final_kernel,
        grid=(pl.cdiv(E, EBLK),),
        in_specs=[
            pl.BlockSpec((EBLK, H), lambda i: (i, 0)),
            pl.BlockSpec((H, H), lambda i: (0, 0)),
            pl.BlockSpec((1, H), lambda i: (0, 0)),
            pl.BlockSpec((EBLK, H), lambda i: (i, 0)),
        ],
        out_specs=pl.BlockSpec((EBLK, H), lambda i: (i, 0)),
        out_shape=jax.ShapeDtypeStruct((E, H), jnp.float32),
    )(edge_feat, WeT, be, d)


def _final_kernel(hp_ref, x_ref, h_ref):
    h_ref[...] = hp_ref[0, :, :] + hp_ref[1, :, :] + x_ref[...]


def _final_add(hpart, x):
    return pl.pallas_call(
        _final_kernel,
        grid=(5,),
        in_specs=[
            pl.BlockSpec((2, 2000, H), lambda i: (0, i, 0)),
            pl.BlockSpec((2000, H), lambda i: (i, 0)),
        ],
        out_specs=pl.BlockSpec((2000, H), lambda i: (i, 0)),
        out_shape=jax.ShapeDtypeStruct((N, H), jnp.float32),
    )(hpart, x)


def _lrelu(v):
    return jnp.where(v >= 0.0, v, 0.01 * v)


def _sc_mesh():
    return plsc.VectorSubcoreMesh(core_axis_name="c", subcore_axis_name="s")


def _sc_scalar_pass(a_pad, b_pad, src, dst, ce1, ce2):
    @functools.partial(
        pl.kernel,
        out_type=[
            jax.ShapeDtypeStruct((E,), jnp.float32),
            jax.ShapeDtypeStruct((E,), jnp.float32),
            jax.ShapeDtypeStruct((NC, L), jnp.float32),
            jax.ShapeDtypeStruct((NW, NP), jnp.float32),
        ],
        mesh=_sc_mesh(),
        compiler_params=pltpu.CompilerParams(needs_layout_passes=False),
        scratch_types=[
            pltpu.VMEM((NP,), jnp.float32),
            pltpu.VMEM((NP,), jnp.float32),
            pltpu.VMEM((EPT,), jnp.int32),
            pltpu.VMEM((EPT,), jnp.float32),
            pltpu.VMEM((NP,), jnp.float32),
            pltpu.VMEM((CH,), jnp.int32),
            pltpu.VMEM((CH,), jnp.float32),
            pltpu.VMEM((CH,), jnp.float32),
            pltpu.VMEM((CH,), jnp.float32),
            pltpu.VMEM((CH,), jnp.float32),
            pltpu.VMEM((L,), jnp.float32),
            pltpu.VMEM((NS, L), jnp.float32),
            pltpu.VMEM_SHARED((NS, L), jnp.float32),
        ],
    )
    def body(a_hbm, b_hbm, src_hbm, dst_hbm, ce1_hbm, ce2_hbm,
             alpha_out, e_out, mc_out, pssum_out,
             a_tab, b_tab, dst_tab, score_tab, ssum_tab,
             src_c, ce1_c, ce2_c, alpha_c, e_c, rmax, maxloc, max_sh):
        c = lax.axis_index("c")
        s = lax.axis_index("s")
        w = s * NC + c
        ebase = w * EPT

        pltpu.sync_copy(a_hbm, a_tab)
        pltpu.sync_copy(b_hbm, b_tab)
        rmax[...] = jnp.full((L,), -3e38, jnp.float32)

        def chunk_body(i, carry):
            off = ebase + i * CH
            pltpu.sync_copy(src_hbm.at[pl.ds(off, CH)], src_c)
            pltpu.sync_copy(dst_hbm.at[pl.ds(off, CH)], dst_tab.at[pl.ds(i * CH, CH)])
            pltpu.sync_copy(ce1_hbm.at[pl.ds(off, CH)], ce1_c)
            pltpu.sync_copy(ce2_hbm.at[pl.ds(off, CH)], ce2_c)

            def grp(g, carry2):
                sl = pl.ds(g * L, L)
                si = src_c[sl]
                di = dst_tab[pl.ds(i * CH + g * L, L)]
                a_s = plsc.load_gather(a_tab, [si])
                a_d = plsc.load_gather(a_tab, [di])
                b_s = plsc.load_gather(b_tab, [si])
                b_d = plsc.load_gather(b_tab, [di])
                c1 = ce1_c[sl]
                c2 = ce2_c[sl]
                z1 = _lrelu(a_s + c1)
                z2 = _lrelu(a_d + c1)
                dz = z1 - z2
                p = jnp.exp(-jnp.abs(dz))
                q = 1.0 / (1.0 + p)
                al = jnp.where(dz >= 0.0, q, p * q)
                sc_ = _lrelu((1.0 + al) * b_s + (2.0 - al) * b_d + c2)
                alpha_c[sl] = al
                score_tab[pl.ds(i * CH + g * L, L)] = sc_
                rmax[...] = jnp.maximum(rmax[...], sc_)
                return carry2

            lax.fori_loop(0, CH // L, grp, 0)
            pltpu.sync_copy(alpha_c, alpha_out.at[pl.ds(off, CH)])
            return carry

        lax.fori_loop(0, EPT // CH, chunk_body, 0)

        pltpu.sync_copy(rmax, max_sh.at[s])
        plsc.subcore_barrier()
        pltpu.sync_copy(max_sh, maxloc)
        mv = maxloc[0, :]
        for k in range(1, NS):
            mv = jnp.maximum(mv, maxloc[k, :])
        mc = jnp.max(mv)
        mcv = jnp.broadcast_to(mc, (L,))

        @pl.when(s == 0)
        def _():
            rmax[...] = mcv
            pltpu.sync_copy(rmax, mc_out.at[c])

        def zr(g, carry):
            ssum_tab[pl.ds(g * L, L)] = jnp.zeros((L,), jnp.float32)
            return carry

        lax.fori_loop(0, NP // L, zr, 0)

        def chunk2(i, carry):
            def grp2(g, carry2):
                sl_t = pl.ds(i * CH + g * L, L)
                ev = jnp.exp(score_tab[sl_t] - mcv)
                e_c[pl.ds(g * L, L)] = ev
                di = dst_tab[sl_t]
                plsc.addupdate_scatter(ssum_tab, [di], ev)
                return carry2

            lax.fori_loop(0, CH // L, grp2, 0)
            pltpu.sync_copy(e_c, e_out.at[pl.ds(ebase + i * CH, CH)])
            return carry

        lax.fori_loop(0, EPT // CH, chunk2, 0)
        pltpu.sync_copy(ssum_tab, pssum_out.at[w])

    return body(a_pad, b_pad, src, dst, ce1, ce2)


def _sc_row_pass(x, src, dst, alpha0, e, mc, pssum):
    @functools.partial(
        pl.kernel,
        out_type=[
            jax.ShapeDtypeStruct((E, H), jnp.float32),
            jax.ShapeDtypeStruct((NC, NP, H), jnp.float32),
        ],
        mesh=_sc_mesh(),
        compiler_params=pltpu.CompilerParams(needs_layout_passes=False),
        scratch_types=[
            pltpu.VMEM((NP,), jnp.float32),
            pltpu.VMEM((NPT,), jnp.float32),
            pltpu.VMEM((NPT,), jnp.float32),
            pltpu.VMEM((NPT,), jnp.float32),
            pltpu.VMEM((NC, L), jnp.float32),
            pltpu.VMEM((CB,), jnp.int32),
            pltpu.VMEM((CB,), jnp.int32),
            pltpu.VMEM((CB,), jnp.float32),
            pltpu.VMEM((CB,), jnp.float32),
            pltpu.VMEM((2, C2), jnp.int32),
            pltpu.VMEM((C, H), jnp.float32),
            pltpu.VMEM((C, H), jnp.float32),
            pltpu.VMEM((C, H), jnp.float32),
            pltpu.VMEM((SROWS, H), jnp.float32),
            pltpu.VMEM_SHARED((NP,), jnp.float32),
            pltpu.VMEM_SHARED((NP, H), jnp.float32),
            pltpu.SemaphoreType.DMA((2,)),
            pltpu.SemaphoreType.DMA((2,)),
            pltpu.SemaphoreType.DMA((2,)),
            pltpu.SemaphoreType.DMA((2,)),
            pltpu.SemaphoreType.DMA((2,)),
        ],
    )
    def body(x_hbm, src_hbm, dst_hbm, al_hbm, e_hbm, mc_hbm, pssum_hbm,
             d_out, hpart_out,
             rinv_tab, row_buf, acc0_b, acc1_b, mcl, src_c, dst_c, al_c,
             bel_c, dst_ch, xs_b, xd_b, d_b, stage, rinv_sh, h_sh,
             lsem_xs, lsem_xd, lsem_dc, wsem_d, wsem_sc):
        c = lax.axis_index("c")
        s = lax.axis_index("s")
        w = s * NC + c
        ebase = w * EPT
        nb = s * NPT

        pltpu.sync_copy(mc_hbm, mcl)
        m0 = mcl[0, :]
        m1 = mcl[1, :]
        mg = jnp.maximum(m0, m1)
        sc0 = jnp.exp(m0 - mg)
        sc1 = jnp.exp(m1 - mg)
        cv = jnp.broadcast_to(c, (L,))
        myscale = jnp.where(cv == 0, sc0, sc1)

        def zacc(g, carry):
            sl = pl.ds(g * L, L)
            acc0_b[sl] = jnp.zeros((L,), jnp.float32)
            acc1_b[sl] = jnp.zeros((L,), jnp.float32)
            return carry

        lax.fori_loop(0, NPT // L, zacc, 0)
        for w2 in range(NW):
            pltpu.sync_copy(pssum_hbm.at[w2, pl.ds(nb, NPT)], row_buf)

            def accg(g, carry, _w2=w2):
                sl = pl.ds(g * L, L)
                if _w2 % NC == 0:
                    acc0_b[sl] = acc0_b[sl] + row_buf[sl]
                else:
                    acc1_b[sl] = acc1_b[sl] + row_buf[sl]
                return carry

            lax.fori_loop(0, NPT // L, accg, 0)

        def cg(g, carry):
            sl = pl.ds(g * L, L)
            tot = acc0_b[sl] * sc0 + acc1_b[sl] * sc1
            row_buf[sl] = 1.0 / jnp.maximum(tot, 1e-16)
            return carry

        lax.fori_loop(0, NPT // L, cg, 0)
        pltpu.sync_copy(row_buf, rinv_sh.at[pl.ds(nb, NPT)])

        def zs(r, carry):
            for j in range(H // L):
                stage[r, pl.ds(j * L, L)] = jnp.zeros((L,), jnp.float32)
            return carry

        lax.fori_loop(0, SROWS, zs, 0)
        for k in range(NPT // SROWS):
            pltpu.sync_copy(stage, h_sh.at[pl.ds(nb + k * SROWS, SROWS), :])

        plsc.subcore_barrier()
        pltpu.sync_copy(rinv_sh, rinv_tab)

        def _load_descs(boff, i):
            p = i % 2
            rows = pl.ds(p * C2, C2)
            off = boff + i * C2
            return [
                pltpu.make_async_copy(
                    x_hbm.at[src_c.at[pl.ds(i * C2, C2)]],
                    xs_b.at[rows, :], lsem_xs.at[p]),
                pltpu.make_async_copy(
                    x_hbm.at[dst_c.at[pl.ds(i * C2, C2)]],
                    xd_b.at[rows, :], lsem_xd.at[p]),
                pltpu.make_async_copy(
                    dst_hbm.at[pl.ds(off, C2)], dst_ch.at[p], lsem_dc.at[p]),
            ]

        def _start_writes(boff, i):
            p = i % 2
            rows = pl.ds(p * C2, C2)
            off = boff + i * C2
            pltpu.make_async_copy(
                d_b.at[rows, :], d_out.at[pl.ds(off, C2), :],
                wsem_d.at[p]).start()
            pltpu.async_copy(
                xs_b.at[rows, :], h_sh.at[dst_ch.at[p]], wsem_sc.at[p],
                add=True)

        def _wait_writes(boff, i):
            p = i % 2
            rows = pl.ds(p * C2, C2)
            off = boff + i * C2
            pltpu.make_async_copy(
                d_b.at[rows, :], d_out.at[pl.ds(off, C2), :],
                wsem_d.at[p]).wait()
            pltpu.make_async_copy(
                xs_b.at[rows, :], h_sh.at[dst_ch.at[p]],
                wsem_sc.at[p]).wait()

        def bat(ib, carry):
            boff = ebase + ib * CB
            pltpu.sync_copy(src_hbm.at[pl.ds(boff, CB)], src_c)
            pltpu.sync_copy(dst_hbm.at[pl.ds(boff, CB)], dst_c)
            pltpu.sync_copy(al_hbm.at[pl.ds(boff, CB)], al_c)
            pltpu.sync_copy(e_hbm.at[pl.ds(boff, CB)], bel_c)

            def pg(g, carry2):
                sl = pl.ds(g * L, L)
                di = dst_c[sl]
                rv = plsc.load_gather(rinv_tab, [di])
                bel_c[sl] = bel_c[sl] * myscale * rv
                return carry2

            lax.fori_loop(0, CB // L, pg, 0)

            for d in _load_descs(boff, 0):
                d.start()

            def half(i, carry2):
                p = i % 2
                for d in _load_descs(boff, i):
                    d.wait()

                @pl.when(i + 1 < HB)
                def _():
                    @pl.when(i >= 1)
                    def _():
                        _wait_writes(boff, i - 1)

                    for d in _load_descs(boff, i + 1):
                        d.start()

                @plsc.parallel_loop(0, C2, 1, unroll=2)
                def _rw(r):
                    rb = p * C2 + r
                    ridx = jnp.broadcast_to(i * C2 + r, (L,)).astype(jnp.int32)
                    alv = plsc.load_gather(al_c, [ridx])
                    blv = plsc.load_gather(bel_c, [ridx])
                    for j in range(H // L):
                        slj = pl.ds(j * L, L)
                        xsv = xs_b[rb, slj]
                        xdv = xd_b[rb, slj]
                        d_b[rb, slj] = alv * (xsv - xdv) + xdv
                        xs_b[rb, slj] = blv * xsv
                _start_writes(boff, i)
                return carry2

            lax.fori_loop(0, HB, half, 0)
            _wait_writes(boff, HB - 2)
            _wait_writes(boff, HB - 1)
            return carry

        lax.fori_loop(0, EPT // CB, bat, 0)
        plsc.subcore_barrier()

        for k in range(NPT // SROWS):
            rows = pl.ds(nb + k * SROWS, SROWS)
            pltpu.sync_copy(h_sh.at[rows, :], stage)
            pltpu.sync_copy(stage, hpart_out.at[c, rows, :])

    return body(x, src, dst, alpha0, e, mc, pssum)


def kernel(node_feat, edge_feat, guild_vec, edge_index, Wn, bn, We, be, Wg, bg, Wna, Wea):
    src = edge_index[0].astype(jnp.int32)
    dst = edge_index[1].astype(jnp.int32)
    wea = Wea[0]
    wna = Wna[0]

    Wab = jnp.stack([wea, wna], axis=1)
    WeAB = We.T @ Wab
    Wg2 = Wg.T @ Wab
    cg = (((be + bg) @ Wab))[None, :]
    cab = jnp.zeros((1, 2), jnp.float32)

    x, a_pad, b_pad = _node_proj(node_feat, Wn.T, bn[None, :], Wab, cab)
    ce1, ce2 = _ce_proj(edge_feat, guild_vec, WeAB, Wg2, cg)

    alpha0, e, mc, pssum = _sc_scalar_pass(a_pad, b_pad, src, dst, ce1, ce2)
    d, hpart = _sc_row_pass(x, src, dst, alpha0, e, mc, pssum)
    ef_new = _edge_final(edge_feat, We.T, be[None, :], d)
    h = _final_add(hpart, x)
    return h, ef_new


# speedup vs baseline: 1.5500x; 1.0021x over previous
import functools

import jax
import jax.numpy as jnp
from jax import lax
from jax.experimental import pallas as pl
from jax.experimental.pallas import tpu as pltpu
from jax.experimental.pallas import tpu_sc as plsc

N = 10000
E = 320000
H = 128

NC = 2
NS = 16
L = 16
NW = NC * NS

NP = 10240
EPT = E // NW
CH = 2000
C = 80
C2 = 40
CB = 400
HB = CB // C2
NPT = NP // NS
SROWS = 16

NBLK = 2048
EBLK = 8192


def _node_proj_kernel(nf_ref, WnT_ref, bn_ref, Wab_ref, cab_ref,
                      x_ref, a_ref, b_ref):
    x = (
        jnp.dot(nf_ref[...], WnT_ref[...], preferred_element_type=jnp.float32)
        + bn_ref[...]
    )
    x_ref[...] = x
    ab = jnp.dot(x, Wab_ref[...], preferred_element_type=jnp.float32)
    a_ref[...] = ab[:, 0]
    b_ref[...] = ab[:, 1]


def _node_proj(node_feat, WnT, bn, Wab, cab):
    return pl.pallas_call(
        _node_proj_kernel,
        grid=(NP // NBLK,),
        in_specs=[
            pl.BlockSpec((NBLK, H), lambda i: (i, 0)),
            pl.BlockSpec((H, H), lambda i: (0, 0)),
            pl.BlockSpec((1, H), lambda i: (0, 0)),
            pl.BlockSpec((H, 2), lambda i: (0, 0)),
            pl.BlockSpec((1, 2), lambda i: (0, 0)),
        ],
        out_specs=[
            pl.BlockSpec((NBLK, H), lambda i: (i, 0)),
            pl.BlockSpec((NBLK,), lambda i: (i,)),
            pl.BlockSpec((NBLK,), lambda i: (i,)),
        ],
        out_shape=[
            jax.ShapeDtypeStruct((N, H), jnp.float32),
            jax.ShapeDtypeStruct((NP,), jnp.float32),
            jax.ShapeDtypeStruct((NP,), jnp.float32),
        ],
    )(node_feat, WnT, bn, Wab, cab)


def _ce_proj_kernel(ef_ref, gv_ref, WeAB_ref, Wg2_ref, cg_ref,
                    ce1_ref, ce2_ref):
    ce_e = jnp.dot(ef_ref[...], WeAB_ref[...], preferred_element_type=jnp.float32)
    ce_g = jnp.dot(gv_ref[...], Wg2_ref[...], preferred_element_type=jnp.float32)
    ce = ce_e + ce_g + cg_ref[...]
    ce1_ref[...] = ce[:, 0]
    ce2_ref[...] = ce[:, 1]


def _ce_proj(edge_feat, guild_vec, WeAB, Wg2, cg):
    return pl.pallas_call(
        _ce_proj_kernel,
        grid=(pl.cdiv(E, EBLK),),
        in_specs=[
            pl.BlockSpec((EBLK, H), lambda i: (i, 0)),
            pl.BlockSpec((EBLK, H), lambda i: (i, 0)),
            pl.BlockSpec((H, 2), lambda i: (0, 0)),
            pl.BlockSpec((H, 2), lambda i: (0, 0)),
            pl.BlockSpec((1, 2), lambda i: (0, 0)),
        ],
        out_specs=[
            pl.BlockSpec((EBLK,), lambda i: (i,)),
            pl.BlockSpec((EBLK,), lambda i: (i,)),
        ],
        out_shape=[
            jax.ShapeDtypeStruct((E,), jnp.float32),
            jax.ShapeDtypeStruct((E,), jnp.float32),
        ],
    )(edge_feat, guild_vec, WeAB, Wg2, cg)


def _edge_final_kernel(ef_ref, WeT_ref, be_ref, d_ref, out_ref):
    out_ref[...] = (
        jnp.dot(ef_ref[...], WeT_ref[...], preferred_element_type=jnp.float32)
        + be_ref[...]
        + d_ref[...]
    )


def _edge_final(edge_feat, WeT, be, d):
    return pl.pallas_call(
        _edge_final_kernel,
        grid=(pl.cdiv(E, EBLK),),
        in_specs=[
            pl.BlockSpec((EBLK, H), lambda i: (i, 0)),
            pl.BlockSpec((H, H), lambda i: (0, 0)),
            pl.BlockSpec((1, H), lambda i: (0, 0)),
            pl.BlockSpec((EBLK, H), lambda i: (i, 0)),
        ],
        out_specs=pl.BlockSpec((EBLK, H), lambda i: (i, 0)),
        out_shape=jax.ShapeDtypeStruct((E, H), jnp.float32),
    )(edge_feat, WeT, be, d)


def _final_kernel(hp_ref, x_ref, h_ref):
    h_ref[...] = hp_ref[0, :, :] + hp_ref[1, :, :] + x_ref[...]


def _final_add(hpart, x):
    return pl.pallas_call(
        _final_kernel,
        grid=(5,),
        in_specs=[
            pl.BlockSpec((2, 2000, H), lambda i: (0, i, 0)),
            pl.BlockSpec((2000, H), lambda i: (i, 0)),
        ],
        out_specs=pl.BlockSpec((2000, H), lambda i: (i, 0)),
        out_shape=jax.ShapeDtypeStruct((N, H), jnp.float32),
    )(hpart, x)


def _lrelu(v):
    return jnp.where(v >= 0.0, v, 0.01 * v)


def _sc_mesh():
    return plsc.VectorSubcoreMesh(core_axis_name="c", subcore_axis_name="s")


def _sc_scalar_pass(a_pad, b_pad, src, dst, ce1, ce2):
    @functools.partial(
        pl.kernel,
        out_type=[
            jax.ShapeDtypeStruct((E,), jnp.float32),
            jax.ShapeDtypeStruct((E,), jnp.float32),
            jax.ShapeDtypeStruct((NC, L), jnp.float32),
            jax.ShapeDtypeStruct((NW, NP), jnp.float32),
        ],
        mesh=_sc_mesh(),
        compiler_params=pltpu.CompilerParams(needs_layout_passes=False),
        scratch_types=[
            pltpu.VMEM((NP,), jnp.float32),
            pltpu.VMEM((NP,), jnp.float32),
            pltpu.VMEM((EPT,), jnp.int32),
            pltpu.VMEM((EPT,), jnp.float32),
            pltpu.VMEM((NP,), jnp.float32),
            pltpu.VMEM((CH,), jnp.int32),
            pltpu.VMEM((CH,), jnp.float32),
            pltpu.VMEM((CH,), jnp.float32),
            pltpu.VMEM((CH,), jnp.float32),
            pltpu.VMEM((CH,), jnp.float32),
            pltpu.VMEM((L,), jnp.float32),
            pltpu.VMEM((NS, L), jnp.float32),
            pltpu.VMEM_SHARED((NS, L), jnp.float32),
        ],
    )
    def body(a_hbm, b_hbm, src_hbm, dst_hbm, ce1_hbm, ce2_hbm,
             alpha_out, e_out, mc_out, pssum_out,
             a_tab, b_tab, dst_tab, score_tab, ssum_tab,
             src_c, ce1_c, ce2_c, alpha_c, e_c, rmax, maxloc, max_sh):
        c = lax.axis_index("c")
        s = lax.axis_index("s")
        w = s * NC + c
        ebase = w * EPT

        pltpu.sync_copy(a_hbm, a_tab)
        pltpu.sync_copy(b_hbm, b_tab)
        rmax[...] = jnp.full((L,), -3e38, jnp.float32)

        def chunk_body(i, carry):
            off = ebase + i * CH
            pltpu.sync_copy(src_hbm.at[pl.ds(off, CH)], src_c)
            pltpu.sync_copy(dst_hbm.at[pl.ds(off, CH)], dst_tab.at[pl.ds(i * CH, CH)])
            pltpu.sync_copy(ce1_hbm.at[pl.ds(off, CH)], ce1_c)
            pltpu.sync_copy(ce2_hbm.at[pl.ds(off, CH)], ce2_c)

            def grp(g, carry2):
                sl = pl.ds(g * L, L)
                si = src_c[sl]
                di = dst_tab[pl.ds(i * CH + g * L, L)]
                a_s = plsc.load_gather(a_tab, [si])
                a_d = plsc.load_gather(a_tab, [di])
                b_s = plsc.load_gather(b_tab, [si])
                b_d = plsc.load_gather(b_tab, [di])
                c1 = ce1_c[sl]
                c2 = ce2_c[sl]
                z1 = _lrelu(a_s + c1)
                z2 = _lrelu(a_d + c1)
                dz = z1 - z2
                p = jnp.exp(-jnp.abs(dz))
                q = 1.0 / (1.0 + p)
                al = jnp.where(dz >= 0.0, q, p * q)
                sc_ = _lrelu((1.0 + al) * b_s + (2.0 - al) * b_d + c2)
                alpha_c[sl] = al
                score_tab[pl.ds(i * CH + g * L, L)] = sc_
                rmax[...] = jnp.maximum(rmax[...], sc_)
                return carry2

            lax.fori_loop(0, CH // L, grp, 0)
            pltpu.sync_copy(alpha_c, alpha_out.at[pl.ds(off, CH)])
            return carry

        lax.fori_loop(0, EPT // CH, chunk_body, 0)

        pltpu.sync_copy(rmax, max_sh.at[s])
        plsc.subcore_barrier()
        pltpu.sync_copy(max_sh, maxloc)
        mv = maxloc[0, :]
        for k in range(1, NS):
            mv = jnp.maximum(mv, maxloc[k, :])
        mc = jnp.max(mv)
        mcv = jnp.broadcast_to(mc, (L,))

        @pl.when(s == 0)
        def _():
            rmax[...] = mcv
            pltpu.sync_copy(rmax, mc_out.at[c])

        def zr(g, carry):
            ssum_tab[pl.ds(g * L, L)] = jnp.zeros((L,), jnp.float32)
            return carry

        lax.fori_loop(0, NP // L, zr, 0)

        def chunk2(i, carry):
            def grp2(g, carry2):
                sl_t = pl.ds(i * CH + g * L, L)
                ev = jnp.exp(score_tab[sl_t] - mcv)
                e_c[pl.ds(g * L, L)] = ev
                di = dst_tab[sl_t]
                plsc.addupdate_scatter(ssum_tab, [di], ev)
                return carry2

            lax.fori_loop(0, CH // L, grp2, 0)
            pltpu.sync_copy(e_c, e_out.at[pl.ds(ebase + i * CH, CH)])
            return carry

        lax.fori_loop(0, EPT // CH, chunk2, 0)
        pltpu.sync_copy(ssum_tab, pssum_out.at[w])

    return body(a_pad, b_pad, src, dst, ce1, ce2)


def _sc_row_pass(x, src, dst, alpha0, e, mc, pssum):
    @functools.partial(
        pl.kernel,
        out_type=[
            jax.ShapeDtypeStruct((E, H), jnp.float32),
            jax.ShapeDtypeStruct((NC, NP, H), jnp.float32),
        ],
        mesh=_sc_mesh(),
        compiler_params=pltpu.CompilerParams(needs_layout_passes=False),
        scratch_types=[
            pltpu.VMEM((NP,), jnp.float32),
            pltpu.VMEM((NPT,), jnp.float32),
            pltpu.VMEM((NPT,), jnp.float32),
            pltpu.VMEM((NPT,), jnp.float32),
            pltpu.VMEM((NC, L), jnp.float32),
            pltpu.VMEM((CB,), jnp.int32),
            pltpu.VMEM((CB,), jnp.int32),
            pltpu.VMEM((CB,), jnp.float32),
            pltpu.VMEM((CB,), jnp.float32),
            pltpu.VMEM((2, C2), jnp.int32),
            pltpu.VMEM((C, H), jnp.float32),
            pltpu.VMEM((C, H), jnp.float32),
            pltpu.VMEM((C, H), jnp.float32),
            pltpu.VMEM((SROWS, H), jnp.float32),
            pltpu.VMEM_SHARED((NP,), jnp.float32),
            pltpu.VMEM_SHARED((NP, H), jnp.float32),
            pltpu.SemaphoreType.DMA((2,)),
            pltpu.SemaphoreType.DMA((2,)),
            pltpu.SemaphoreType.DMA((2,)),
            pltpu.SemaphoreType.DMA((2,)),
            pltpu.SemaphoreType.DMA((2,)),
        ],
    )
    def body(x_hbm, src_hbm, dst_hbm, al_hbm, e_hbm, mc_hbm, pssum_hbm,
             d_out, hpart_out,
             rinv_tab, row_buf, acc0_b, acc1_b, mcl, src_c, dst_c, al_c,
             bel_c, dst_ch, xs_b, xd_b, d_b, stage, rinv_sh, h_sh,
             lsem_xs, lsem_xd, lsem_dc, wsem_d, wsem_sc):
        c = lax.axis_index("c")
        s = lax.axis_index("s")
        w = s * NC + c
        ebase = w * EPT
        nb = s * NPT

        pltpu.sync_copy(mc_hbm, mcl)
        m0 = mcl[0, :]
        m1 = mcl[1, :]
        mg = jnp.maximum(m0, m1)
        sc0 = jnp.exp(m0 - mg)
        sc1 = jnp.exp(m1 - mg)
        cv = jnp.broadcast_to(c, (L,))
        myscale = jnp.where(cv == 0, sc0, sc1)

        def zacc(g, carry):
            sl = pl.ds(g * L, L)
            acc0_b[sl] = jnp.zeros((L,), jnp.float32)
            acc1_b[sl] = jnp.zeros((L,), jnp.float32)
            return carry

        lax.fori_loop(0, NPT // L, zacc, 0)
        for w2 in range(NW):
            pltpu.sync_copy(pssum_hbm.at[w2, pl.ds(nb, NPT)], row_buf)

            def accg(g, carry, _w2=w2):
                sl = pl.ds(g * L, L)
                if _w2 % NC == 0:
                    acc0_b[sl] = acc0_b[sl] + row_buf[sl]
                else:
                    acc1_b[sl] = acc1_b[sl] + row_buf[sl]
                return carry

            lax.fori_loop(0, NPT // L, accg, 0)

        def cg(g, carry):
            sl = pl.ds(g * L, L)
            tot = acc0_b[sl] * sc0 + acc1_b[sl] * sc1
            row_buf[sl] = 1.0 / jnp.maximum(tot, 1e-16)
            return carry

        lax.fori_loop(0, NPT // L, cg, 0)
        pltpu.sync_copy(row_buf, rinv_sh.at[pl.ds(nb, NPT)])

        def zs(r, carry):
            for j in range(H // L):
                stage[r, pl.ds(j * L, L)] = jnp.zeros((L,), jnp.float32)
            return carry

        lax.fori_loop(0, SROWS, zs, 0)
        for k in range(NPT // SROWS):
            pltpu.sync_copy(stage, h_sh.at[pl.ds(nb + k * SROWS, SROWS), :])

        plsc.subcore_barrier()
        pltpu.sync_copy(rinv_sh, rinv_tab)

        def _load_descs(boff, i):
            p = i % 2
            rows = pl.ds(p * C2, C2)
            off = boff + i * C2
            return [
                pltpu.make_async_copy(
                    x_hbm.at[src_c.at[pl.ds(i * C2, C2)]],
                    xs_b.at[rows, :], lsem_xs.at[p]),
                pltpu.make_async_copy(
                    x_hbm.at[dst_c.at[pl.ds(i * C2, C2)]],
                    xd_b.at[rows, :], lsem_xd.at[p]),
                pltpu.make_async_copy(
                    dst_hbm.at[pl.ds(off, C2)], dst_ch.at[p], lsem_dc.at[p]),
            ]

        def _start_writes(boff, i):
            p = i % 2
            rows = pl.ds(p * C2, C2)
            off = boff + i * C2
            pltpu.make_async_copy(
                d_b.at[rows, :], d_out.at[pl.ds(off, C2), :],
                wsem_d.at[p]).start()
            pltpu.async_copy(
                xs_b.at[rows, :], h_sh.at[dst_ch.at[p]], wsem_sc.at[p],
                add=True)

        def _wait_writes(boff, i):
            p = i % 2
            rows = pl.ds(p * C2, C2)
            off = boff + i * C2
            pltpu.make_async_copy(
                d_b.at[rows, :], d_out.at[pl.ds(off, C2), :],
                wsem_d.at[p]).wait()
            pltpu.make_async_copy(
                xs_b.at[rows, :], h_sh.at[dst_ch.at[p]],
                wsem_sc.at[p]).wait()

        def bat(ib, carry):
            boff = ebase + ib * CB
            pltpu.sync_copy(src_hbm.at[pl.ds(boff, CB)], src_c)
            pltpu.sync_copy(dst_hbm.at[pl.ds(boff, CB)], dst_c)
            pltpu.sync_copy(al_hbm.at[pl.ds(boff, CB)], al_c)
            pltpu.sync_copy(e_hbm.at[pl.ds(boff, CB)], bel_c)

            @plsc.parallel_loop(0, CB // L, 1, unroll=2)
            def _pg(g):
                sl = pl.ds(g * L, L)
                di = dst_c[sl]
                rv = plsc.load_gather(rinv_tab, [di])
                bel_c[sl] = bel_c[sl] * myscale * rv

            for d in _load_descs(boff, 0):
                d.start()

            def half(i, carry2):
                p = i % 2
                for d in _load_descs(boff, i):
                    d.wait()

                @pl.when(i + 1 < HB)
                def _():
                    @pl.when(i >= 1)
                    def _():
                        _wait_writes(boff, i - 1)

                    for d in _load_descs(boff, i + 1):
                        d.start()

                @plsc.parallel_loop(0, C2, 1, unroll=4)
                def _rw(r):
                    rb = p * C2 + r
                    ridx = jnp.broadcast_to(i * C2 + r, (L,)).astype(jnp.int32)
                    alv = plsc.load_gather(al_c, [ridx])
                    blv = plsc.load_gather(bel_c, [ridx])
                    for j in range(H // L):
                        slj = pl.ds(j * L, L)
                        xsv = xs_b[rb, slj]
                        xdv = xd_b[rb, slj]
                        d_b[rb, slj] = alv * (xsv - xdv) + xdv
                        xs_b[rb, slj] = blv * xsv
                _start_writes(boff, i)
                return carry2

            lax.fori_loop(0, HB, half, 0)
            _wait_writes(boff, HB - 2)
            _wait_writes(boff, HB - 1)
            return carry

        lax.fori_loop(0, EPT // CB, bat, 0)
        plsc.subcore_barrier()

        for k in range(NPT // SROWS):
            rows = pl.ds(nb + k * SROWS, SROWS)
            pltpu.sync_copy(h_sh.at[rows, :], stage)
            pltpu.sync_copy(stage, hpart_out.at[c, rows, :])

    return body(x, src, dst, alpha0, e, mc, pssum)


def kernel(node_feat, edge_feat, guild_vec, edge_index, Wn, bn, We, be, Wg, bg, Wna, Wea):
    src = edge_index[0].astype(jnp.int32)
    dst = edge_index[1].astype(jnp.int32)
    wea = Wea[0]
    wna = Wna[0]

    Wab = jnp.stack([wea, wna], axis=1)
    WeAB = We.T @ Wab
    Wg2 = Wg.T @ Wab
    cg = (((be + bg) @ Wab))[None, :]
    cab = jnp.zeros((1, 2), jnp.float32)

    x, a_pad, b_pad = _node_proj(node_feat, Wn.T, bn[None, :], Wab, cab)
    ce1, ce2 = _ce_proj(edge_feat, guild_vec, WeAB, Wg2, cg)

    alpha0, e, mc, pssum = _sc_scalar_pass(a_pad, b_pad, src, dst, ce1, ce2)
    d, hpart = _sc_row_pass(x, src, dst, alpha0, e, mc, pssum)
    ef_new = _edge_final(edge_feat, We.T, be[None, :], d)
    h = _final_add(hpart, x)
    return h, ef_new


# speedup vs baseline: 1.5510x; 1.0007x over previous
import functools

import jax
import jax.numpy as jnp
from jax import lax
from jax.experimental import pallas as pl
from jax.experimental.pallas import tpu as pltpu
from jax.experimental.pallas import tpu_sc as plsc

N = 10000
E = 320000
H = 128

NC = 2
NS = 16
L = 16
NW = NC * NS

NP = 10240
EPT = E // NW
CH = 2000
C = 80
C2 = 40
CB = 400
HB = CB // C2
NPT = NP // NS
SROWS = 16

NBLK = 2048
EBLK = 8192


def _node_proj_kernel(nf_ref, WnT_ref, bn_ref, Wab_ref, cab_ref,
                      x_ref, a_ref, b_ref):
    x = (
        jnp.dot(nf_ref[...], WnT_ref[...], preferred_element_type=jnp.float32)
        + bn_ref[...]
    )
    x_ref[...] = x
    ab = jnp.dot(x, Wab_ref[...], preferred_element_type=jnp.float32)
    a_ref[...] = ab[:, 0]
    b_ref[...] = ab[:, 1]


def _node_proj(node_feat, WnT, bn, Wab, cab):
    return pl.pallas_call(
        _node_proj_kernel,
        grid=(NP // NBLK,),
        in_specs=[
            pl.BlockSpec((NBLK, H), lambda i: (i, 0)),
            pl.BlockSpec((H, H), lambda i: (0, 0)),
            pl.BlockSpec((1, H), lambda i: (0, 0)),
            pl.BlockSpec((H, 2), lambda i: (0, 0)),
            pl.BlockSpec((1, 2), lambda i: (0, 0)),
        ],
        out_specs=[
            pl.BlockSpec((NBLK, H), lambda i: (i, 0)),
            pl.BlockSpec((NBLK,), lambda i: (i,)),
            pl.BlockSpec((NBLK,), lambda i: (i,)),
        ],
        out_shape=[
            jax.ShapeDtypeStruct((N, H), jnp.float32),
            jax.ShapeDtypeStruct((NP,), jnp.float32),
            jax.ShapeDtypeStruct((NP,), jnp.float32),
        ],
    )(node_feat, WnT, bn, Wab, cab)


def _ce_proj_kernel(ef_ref, gv_ref, WeAB_ref, Wg2_ref, cg_ref,
                    ce1_ref, ce2_ref):
    ce_e = jnp.dot(ef_ref[...], WeAB_ref[...], preferred_element_type=jnp.float32)
    ce_g = jnp.dot(gv_ref[...], Wg2_ref[...], preferred_element_type=jnp.float32)
    ce = ce_e + ce_g + cg_ref[...]
    ce1_ref[...] = ce[:, 0]
    ce2_ref[...] = ce[:, 1]


def _ce_proj(edge_feat, guild_vec, WeAB, Wg2, cg):
    return pl.pallas_call(
        _ce_proj_kernel,
        grid=(pl.cdiv(E, EBLK),),
        in_specs=[
            pl.BlockSpec((EBLK, H), lambda i: (i, 0)),
            pl.BlockSpec((EBLK, H), lambda i: (i, 0)),
            pl.BlockSpec((H, 2), lambda i: (0, 0)),
            pl.BlockSpec((H, 2), lambda i: (0, 0)),
            pl.BlockSpec((1, 2), lambda i: (0, 0)),
        ],
        out_specs=[
            pl.BlockSpec((EBLK,), lambda i: (i,)),
            pl.BlockSpec((EBLK,), lambda i: (i,)),
        ],
        out_shape=[
            jax.ShapeDtypeStruct((E,), jnp.float32),
            jax.ShapeDtypeStruct((E,), jnp.float32),
        ],
    )(edge_feat, guild_vec, WeAB, Wg2, cg)


def _edge_final_kernel(ef_ref, WeT_ref, be_ref, d_ref, out_ref):
    out_ref[...] = (
        jnp.dot(ef_ref[...], WeT_ref[...], preferred_element_type=jnp.float32)
        + be_ref[...]
        + d_ref[...]
    )


def _edge_final(edge_feat, WeT, be, d):
    return pl.pallas_call(
        _edge_final_kernel,
        grid=(pl.cdiv(E, EBLK),),
        in_specs=[
            pl.BlockSpec((EBLK, H), lambda i: (i, 0)),
            pl.BlockSpec((H, H), lambda i: (0, 0)),
            pl.BlockSpec((1, H), lambda i: (0, 0)),
            pl.BlockSpec((EBLK, H), lambda i: (i, 0)),
        ],
        out_specs=pl.BlockSpec((EBLK, H), lambda i: (i, 0)),
        out_shape=jax.ShapeDtypeStruct((E, H), jnp.float32),
    )(edge_feat, WeT, be, d)


def _final_kernel(hp_ref, x_ref, h_ref):
    h_ref[...] = hp_ref[0, :, :] + hp_ref[1, :, :] + x_ref[...]


def _final_add(hpart, x):
    return pl.pallas_call(
        _final_kernel,
        grid=(5,),
        in_specs=[
            pl.BlockSpec((2, 2000, H), lambda i: (0, i, 0)),
            pl.BlockSpec((2000, H), lambda i: (i, 0)),
        ],
        out_specs=pl.BlockSpec((2000, H), lambda i: (i, 0)),
        out_shape=jax.ShapeDtypeStruct((N, H), jnp.float32),
    )(hpart, x)


def _lrelu(v):
    return jnp.where(v >= 0.0, v, 0.01 * v)


def _sc_mesh():
    return plsc.VectorSubcoreMesh(core_axis_name="c", subcore_axis_name="s")


def _sc_scalar_pass(a_pad, b_pad, src, dst, ce1, ce2):
    @functools.partial(
        pl.kernel,
        out_type=[
            jax.ShapeDtypeStruct((E,), jnp.float32),
            jax.ShapeDtypeStruct((E,), jnp.float32),
            jax.ShapeDtypeStruct((NC, L), jnp.float32),
            jax.ShapeDtypeStruct((NW, NP), jnp.float32),
        ],
        mesh=_sc_mesh(),
        compiler_params=pltpu.CompilerParams(needs_layout_passes=False),
        scratch_types=[
            pltpu.VMEM((NP,), jnp.float32),
            pltpu.VMEM((NP,), jnp.float32),
            pltpu.VMEM((EPT,), jnp.int32),
            pltpu.VMEM((EPT,), jnp.float32),
            pltpu.VMEM((NP,), jnp.float32),
            pltpu.VMEM((CH,), jnp.int32),
            pltpu.VMEM((CH,), jnp.float32),
            pltpu.VMEM((CH,), jnp.float32),
            pltpu.VMEM((CH,), jnp.float32),
            pltpu.VMEM((CH,), jnp.float32),
            pltpu.VMEM((L,), jnp.float32),
            pltpu.VMEM((NS, L), jnp.float32),
            pltpu.VMEM_SHARED((NS, L), jnp.float32),
        ],
    )
    def body(a_hbm, b_hbm, src_hbm, dst_hbm, ce1_hbm, ce2_hbm,
             alpha_out, e_out, mc_out, pssum_out,
             a_tab, b_tab, dst_tab, score_tab, ssum_tab,
             src_c, ce1_c, ce2_c, alpha_c, e_c, rmax, maxloc, max_sh):
        c = lax.axis_index("c")
        s = lax.axis_index("s")
        w = s * NC + c
        ebase = w * EPT

        pltpu.sync_copy(a_hbm, a_tab)
        pltpu.sync_copy(b_hbm, b_tab)
        rmax[...] = jnp.full((L,), -3e38, jnp.float32)

        def chunk_body(i, carry):
            off = ebase + i * CH
            pltpu.sync_copy(src_hbm.at[pl.ds(off, CH)], src_c)
            pltpu.sync_copy(dst_hbm.at[pl.ds(off, CH)], dst_tab.at[pl.ds(i * CH, CH)])
            pltpu.sync_copy(ce1_hbm.at[pl.ds(off, CH)], ce1_c)
            pltpu.sync_copy(ce2_hbm.at[pl.ds(off, CH)], ce2_c)

            def grp(g, carry2):
                sl = pl.ds(g * L, L)
                si = src_c[sl]
                di = dst_tab[pl.ds(i * CH + g * L, L)]
                a_s = plsc.load_gather(a_tab, [si])
                a_d = plsc.load_gather(a_tab, [di])
                b_s = plsc.load_gather(b_tab, [si])
                b_d = plsc.load_gather(b_tab, [di])
                c1 = ce1_c[sl]
                c2 = ce2_c[sl]
                z1 = _lrelu(a_s + c1)
                z2 = _lrelu(a_d + c1)
                dz = z1 - z2
                p = jnp.exp(-jnp.abs(dz))
                q = 1.0 / (1.0 + p)
                al = jnp.where(dz >= 0.0, q, p * q)
                sc_ = _lrelu((1.0 + al) * b_s + (2.0 - al) * b_d + c2)
                alpha_c[sl] = al
                score_tab[pl.ds(i * CH + g * L, L)] = sc_
                rmax[...] = jnp.maximum(rmax[...], sc_)
                return carry2

            lax.fori_loop(0, CH // L, grp, 0)
            pltpu.sync_copy(alpha_c, alpha_out.at[pl.ds(off, CH)])
            return carry

        lax.fori_loop(0, EPT // CH, chunk_body, 0)

        pltpu.sync_copy(rmax, max_sh.at[s])
        plsc.subcore_barrier()
        pltpu.sync_copy(max_sh, maxloc)
        mv = maxloc[0, :]
        for k in range(1, NS):
            mv = jnp.maximum(mv, maxloc[k, :])
        mc = jnp.max(mv)
        mcv = jnp.broadcast_to(mc, (L,))

        @pl.when(s == 0)
        def _():
            rmax[...] = mcv
            pltpu.sync_copy(rmax, mc_out.at[c])

        def zr(g, carry):
            ssum_tab[pl.ds(g * L, L)] = jnp.zeros((L,), jnp.float32)
            return carry

        lax.fori_loop(0, NP // L, zr, 0)

        def chunk2(i, carry):
            def grp2(g, carry2):
                sl_t = pl.ds(i * CH + g * L, L)
                ev = jnp.exp(score_tab[sl_t] - mcv)
                e_c[pl.ds(g * L, L)] = ev
                di = dst_tab[sl_t]
                plsc.addupdate_scatter(ssum_tab, [di], ev)
                return carry2

            lax.fori_loop(0, CH // L, grp2, 0)
            pltpu.sync_copy(e_c, e_out.at[pl.ds(ebase + i * CH, CH)])
            return carry

        lax.fori_loop(0, EPT // CH, chunk2, 0)
        pltpu.sync_copy(ssum_tab, pssum_out.at[w])

    return body(a_pad, b_pad, src, dst, ce1, ce2)


def _sc_row_pass(x, src, dst, alpha0, e, mc, pssum):
    @functools.partial(
        pl.kernel,
        out_type=[
            jax.ShapeDtypeStruct((E, H), jnp.float32),
            jax.ShapeDtypeStruct((NC, NP, H), jnp.float32),
        ],
        mesh=_sc_mesh(),
        compiler_params=pltpu.CompilerParams(needs_layout_passes=False),
        scratch_types=[
            pltpu.VMEM((NP,), jnp.float32),
            pltpu.VMEM((NPT,), jnp.float32),
            pltpu.VMEM((NPT,), jnp.float32),
            pltpu.VMEM((NPT,), jnp.float32),
            pltpu.VMEM((NC, L), jnp.float32),
            pltpu.VMEM((CB,), jnp.int32),
            pltpu.VMEM((CB,), jnp.int32),
            pltpu.VMEM((CB,), jnp.float32),
            pltpu.VMEM((CB,), jnp.float32),
            pltpu.VMEM((2, C2), jnp.int32),
            pltpu.VMEM((C, H), jnp.float32),
            pltpu.VMEM((C, H), jnp.float32),
            pltpu.VMEM((C, H), jnp.float32),
            pltpu.VMEM((SROWS, H), jnp.float32),
            pltpu.VMEM_SHARED((NP,), jnp.float32),
            pltpu.VMEM_SHARED((NP, H), jnp.float32),
            pltpu.SemaphoreType.DMA((2,)),
            pltpu.SemaphoreType.DMA((2,)),
            pltpu.SemaphoreType.DMA((2,)),
            pltpu.SemaphoreType.DMA((2,)),
            pltpu.SemaphoreType.DMA((2,)),
        ],
    )
    def body(x_hbm, src_hbm, dst_hbm, al_hbm, e_hbm, mc_hbm, pssum_hbm,
             d_out, hpart_out,
             rinv_tab, row_buf, acc0_b, acc1_b, mcl, src_c, dst_c, al_c,
             bel_c, dst_ch, xs_b, xd_b, d_b, stage, rinv_sh, h_sh,
             lsem_xs, lsem_xd, lsem_dc, wsem_d, wsem_sc):
        c = lax.axis_index("c")
        s = lax.axis_index("s")
        w = s * NC + c
        ebase = w * EPT
        nb = s * NPT

        pltpu.sync_copy(mc_hbm, mcl)
        m0 = mcl[0, :]
        m1 = mcl[1, :]
        mg = jnp.maximum(m0, m1)
        sc0 = jnp.exp(m0 - mg)
        sc1 = jnp.exp(m1 - mg)
        cv = jnp.broadcast_to(c, (L,))
        myscale = jnp.where(cv == 0, sc0, sc1)

        def zacc(g, carry):
            sl = pl.ds(g * L, L)
            acc0_b[sl] = jnp.zeros((L,), jnp.float32)
            acc1_b[sl] = jnp.zeros((L,), jnp.float32)
            return carry

        lax.fori_loop(0, NPT // L, zacc, 0)
        for w2 in range(NW):
            pltpu.sync_copy(pssum_hbm.at[w2, pl.ds(nb, NPT)], row_buf)

            def accg(g, carry, _w2=w2):
                sl = pl.ds(g * L, L)
                if _w2 % NC == 0:
                    acc0_b[sl] = acc0_b[sl] + row_buf[sl]
                else:
                    acc1_b[sl] = acc1_b[sl] + row_buf[sl]
                return carry

            lax.fori_loop(0, NPT // L, accg, 0)

        def cg(g, carry):
            sl = pl.ds(g * L, L)
            tot = acc0_b[sl] * sc0 + acc1_b[sl] * sc1
            row_buf[sl] = 1.0 / jnp.maximum(tot, 1e-16)
            return carry

        lax.fori_loop(0, NPT // L, cg, 0)
        pltpu.sync_copy(row_buf, rinv_sh.at[pl.ds(nb, NPT)])

        def zs(r, carry):
            for j in range(H // L):
                stage[r, pl.ds(j * L, L)] = jnp.zeros((L,), jnp.float32)
            return carry

        lax.fori_loop(0, SROWS, zs, 0)
        for k in range(NPT // SROWS):
            pltpu.sync_copy(stage, h_sh.at[pl.ds(nb + k * SROWS, SROWS), :])

        plsc.subcore_barrier()
        pltpu.sync_copy(rinv_sh, rinv_tab)

        def _load_descs(boff, i):
            p = i % 2
            rows = pl.ds(p * C2, C2)
            off = boff + i * C2
            return [
                pltpu.make_async_copy(
                    x_hbm.at[src_c.at[pl.ds(i * C2, C2)]],
                    xs_b.at[rows, :], lsem_xs.at[p]),
                pltpu.make_async_copy(
                    x_hbm.at[dst_c.at[pl.ds(i * C2, C2)]],
                    xd_b.at[rows, :], lsem_xd.at[p]),
                pltpu.make_async_copy(
                    dst_hbm.at[pl.ds(off, C2)], dst_ch.at[p], lsem_dc.at[p]),
            ]

        def _start_writes(boff, i):
            p = i % 2
            rows = pl.ds(p * C2, C2)
            off = boff + i * C2
            pltpu.make_async_copy(
                d_b.at[rows, :], d_out.at[pl.ds(off, C2), :],
                wsem_d.at[p]).start()
            pltpu.async_copy(
                xs_b.at[rows, :], h_sh.at[dst_ch.at[p]], wsem_sc.at[p],
                add=True)

        def _wait_writes(boff, i):
            p = i % 2
            rows = pl.ds(p * C2, C2)
            off = boff + i * C2
            pltpu.make_async_copy(
                d_b.at[rows, :], d_out.at[pl.ds(off, C2), :],
                wsem_d.at[p]).wait()
            pltpu.make_async_copy(
                xs_b.at[rows, :], h_sh.at[dst_ch.at[p]],
                wsem_sc.at[p]).wait()

        def bat(ib, carry):
            boff = ebase + ib * CB
            pltpu.sync_copy(src_hbm.at[pl.ds(boff, CB)], src_c)
            pltpu.sync_copy(dst_hbm.at[pl.ds(boff, CB)], dst_c)
            pltpu.sync_copy(al_hbm.at[pl.ds(boff, CB)], al_c)
            pltpu.sync_copy(e_hbm.at[pl.ds(boff, CB)], bel_c)

            @plsc.parallel_loop(0, CB // L, 1, unroll=2)
            def _pg(g):
                sl = pl.ds(g * L, L)
                di = dst_c[sl]
                rv = plsc.load_gather(rinv_tab, [di])
                bel_c[sl] = bel_c[sl] * myscale * rv

            for d in _load_descs(boff, 0):
                d.start()

            def half(i, carry2):
                p = i % 2
                for d in _load_descs(boff, i):
                    d.wait()

                @pl.when(i + 1 < HB)
                def _():
                    @pl.when(i >= 1)
                    def _():
                        _wait_writes(boff, i - 1)

                    for d in _load_descs(boff, i + 1):
                        d.start()

                @plsc.parallel_loop(0, C2, 1, unroll=8)
                def _rw(r):
                    rb = p * C2 + r
                    ridx = jnp.broadcast_to(i * C2 + r, (L,)).astype(jnp.int32)
                    alv = plsc.load_gather(al_c, [ridx])
                    blv = plsc.load_gather(bel_c, [ridx])
                    for j in range(H // L):
                        slj = pl.ds(j * L, L)
                        xsv = xs_b[rb, slj]
                        xdv = xd_b[rb, slj]
                        d_b[rb, slj] = alv * (xsv - xdv) + xdv
                        xs_b[rb, slj] = blv * xsv
                _start_writes(boff, i)
                return carry2

            lax.fori_loop(0, HB, half, 0)
            _wait_writes(boff, HB - 2)
            _wait_writes(boff, HB - 1)
            return carry

        lax.fori_loop(0, EPT // CB, bat, 0)
        plsc.subcore_barrier()

        for k in range(NPT // SROWS):
            rows = pl.ds(nb + k * SROWS, SROWS)
            pltpu.sync_copy(h_sh.at[rows, :], stage)
            pltpu.sync_copy(stage, hpart_out.at[c, rows, :])

    return body(x, src, dst, alpha0, e, mc, pssum)


def kernel(node_feat, edge_feat, guild_vec, edge_index, Wn, bn, We, be, Wg, bg, Wna, Wea):
    src = edge_index[0].astype(jnp.int32)
    dst = edge_index[1].astype(jnp.int32)
    wea = Wea[0]
    wna = Wna[0]

    Wab = jnp.stack([wea, wna], axis=1)
    WeAB = We.T @ Wab
    Wg2 = Wg.T @ Wab
    cg = (((be + bg) @ Wab))[None, :]
    cab = jnp.zeros((1, 2), jnp.float32)

    x, a_pad, b_pad = _node_proj(node_feat, Wn.T, bn[None, :], Wab, cab)
    ce1, ce2 = _ce_proj(edge_feat, guild_vec, WeAB, Wg2, cg)

    alpha0, e, mc, pssum = _sc_scalar_pass(a_pad, b_pad, src, dst, ce1, ce2)
    d, hpart = _sc_row_pass(x, src, dst, alpha0, e, mc, pssum)
    ef_new = _edge_final(edge_feat, We.T, be[None, :], d)
    h = _final_add(hpart, x)
    return h, ef_new


# speedup vs baseline: 1.6477x; 1.0624x over previous
import functools

import jax
import jax.numpy as jnp
from jax import lax
from jax.experimental import pallas as pl
from jax.experimental.pallas import tpu as pltpu
from jax.experimental.pallas import tpu_sc as plsc

N = 10000
E = 320000
H = 128

NC = 2
NS = 16
L = 16
NW = NC * NS

NP = 10240
EPT = E // NW
CH = 2000
C = 64
C2 = 16
RING = 4
CB = 400
HB = CB // C2
NPT = NP // NS
SROWS = 16

NBLK = 2048
EBLK = 8192


def _node_proj_kernel(nf_ref, WnT_ref, bn_ref, Wab_ref, cab_ref,
                      x_ref, a_ref, b_ref):
    x = (
        jnp.dot(nf_ref[...], WnT_ref[...], preferred_element_type=jnp.float32)
        + bn_ref[...]
    )
    x_ref[...] = x
    ab = jnp.dot(x, Wab_ref[...], preferred_element_type=jnp.float32)
    a_ref[...] = ab[:, 0]
    b_ref[...] = ab[:, 1]


def _node_proj(node_feat, WnT, bn, Wab, cab):
    return pl.pallas_call(
        _node_proj_kernel,
        grid=(NP // NBLK,),
        in_specs=[
            pl.BlockSpec((NBLK, H), lambda i: (i, 0)),
            pl.BlockSpec((H, H), lambda i: (0, 0)),
            pl.BlockSpec((1, H), lambda i: (0, 0)),
            pl.BlockSpec((H, 2), lambda i: (0, 0)),
            pl.BlockSpec((1, 2), lambda i: (0, 0)),
        ],
        out_specs=[
            pl.BlockSpec((NBLK, H), lambda i: (i, 0)),
            pl.BlockSpec((NBLK,), lambda i: (i,)),
            pl.BlockSpec((NBLK,), lambda i: (i,)),
        ],
        out_shape=[
            jax.ShapeDtypeStruct((N, H), jnp.float32),
            jax.ShapeDtypeStruct((NP,), jnp.float32),
            jax.ShapeDtypeStruct((NP,), jnp.float32),
        ],
    )(node_feat, WnT, bn, Wab, cab)


def _ce_proj_kernel(ef_ref, gv_ref, WeAB_ref, Wg2_ref, cg_ref,
                    ce1_ref, ce2_ref):
    ce_e = jnp.dot(ef_ref[...], WeAB_ref[...], preferred_element_type=jnp.float32)
    ce_g = jnp.dot(gv_ref[...], Wg2_ref[...], preferred_element_type=jnp.float32)
    ce = ce_e + ce_g + cg_ref[...]
    ce1_ref[...] = ce[:, 0]
    ce2_ref[...] = ce[:, 1]


def _ce_proj(edge_feat, guild_vec, WeAB, Wg2, cg):
    return pl.pallas_call(
        _ce_proj_kernel,
        grid=(pl.cdiv(E, EBLK),),
        in_specs=[
            pl.BlockSpec((EBLK, H), lambda i: (i, 0)),
            pl.BlockSpec((EBLK, H), lambda i: (i, 0)),
            pl.BlockSpec((H, 2), lambda i: (0, 0)),
            pl.BlockSpec((H, 2), lambda i: (0, 0)),
            pl.BlockSpec((1, 2), lambda i: (0, 0)),
        ],
        out_specs=[
            pl.BlockSpec((EBLK,), lambda i: (i,)),
            pl.BlockSpec((EBLK,), lambda i: (i,)),
        ],
        out_shape=[
            jax.ShapeDtypeStruct((E,), jnp.float32),
            jax.ShapeDtypeStruct((E,), jnp.float32),
        ],
    )(edge_feat, guild_vec, WeAB, Wg2, cg)


def _edge_final_kernel(ef_ref, WeT_ref, be_ref, d_ref, out_ref):
    out_ref[...] = (
        jnp.dot(ef_ref[...], WeT_ref[...], preferred_element_type=jnp.float32)
        + be_ref[...]
        + d_ref[...]
    )


def _edge_final(edge_feat, WeT, be, d):
    return pl.pallas_call(
        _edge_final_kernel,
        grid=(pl.cdiv(E, EBLK),),
        in_specs=[
            pl.BlockSpec((EBLK, H), lambda i: (i, 0)),
            pl.BlockSpec((H, H), lambda i: (0, 0)),
            pl.BlockSpec((1, H), lambda i: (0, 0)),
            pl.BlockSpec((EBLK, H), lambda i: (i, 0)),
        ],
        out_specs=pl.BlockSpec((EBLK, H), lambda i: (i, 0)),
        out_shape=jax.ShapeDtypeStruct((E, H), jnp.float32),
    )(edge_feat, WeT, be, d)


def _final_kernel(hp_ref, x_ref, h_ref):
    h_ref[...] = hp_ref[0, :, :] + hp_ref[1, :, :] + x_ref[...]


def _final_add(hpart, x):
    return pl.pallas_call(
        _final_kernel,
        grid=(5,),
        in_specs=[
            pl.BlockSpec((2, 2000, H), lambda i: (0, i, 0)),
            pl.BlockSpec((2000, H), lambda i: (i, 0)),
        ],
        out_specs=pl.BlockSpec((2000, H), lambda i: (i, 0)),
        out_shape=jax.ShapeDtypeStruct((N, H), jnp.float32),
    )(hpart, x)


def _lrelu(v):
    return jnp.where(v >= 0.0, v, 0.01 * v)


def _sc_mesh():
    return plsc.VectorSubcoreMesh(core_axis_name="c", subcore_axis_name="s")


def _sc_scalar_pass(a_pad, b_pad, src, dst, ce1, ce2):
    @functools.partial(
        pl.kernel,
        out_type=[
            jax.ShapeDtypeStruct((E,), jnp.float32),
            jax.ShapeDtypeStruct((E,), jnp.float32),
            jax.ShapeDtypeStruct((NC, L), jnp.float32),
            jax.ShapeDtypeStruct((NW, NP), jnp.float32),
        ],
        mesh=_sc_mesh(),
        compiler_params=pltpu.CompilerParams(needs_layout_passes=False),
        scratch_types=[
            pltpu.VMEM((NP,), jnp.float32),
            pltpu.VMEM((NP,), jnp.float32),
            pltpu.VMEM((EPT,), jnp.int32),
            pltpu.VMEM((EPT,), jnp.float32),
            pltpu.VMEM((NP,), jnp.float32),
            pltpu.VMEM((CH,), jnp.int32),
            pltpu.VMEM((CH,), jnp.float32),
            pltpu.VMEM((CH,), jnp.float32),
            pltpu.VMEM((CH,), jnp.float32),
            pltpu.VMEM((CH,), jnp.float32),
            pltpu.VMEM((L,), jnp.float32),
            pltpu.VMEM((NS, L), jnp.float32),
            pltpu.VMEM_SHARED((NS, L), jnp.float32),
        ],
    )
    def body(a_hbm, b_hbm, src_hbm, dst_hbm, ce1_hbm, ce2_hbm,
             alpha_out, e_out, mc_out, pssum_out,
             a_tab, b_tab, dst_tab, score_tab, ssum_tab,
             src_c, ce1_c, ce2_c, alpha_c, e_c, rmax, maxloc, max_sh):
        c = lax.axis_index("c")
        s = lax.axis_index("s")
        w = s * NC + c
        ebase = w * EPT

        pltpu.sync_copy(a_hbm, a_tab)
        pltpu.sync_copy(b_hbm, b_tab)
        rmax[...] = jnp.full((L,), -3e38, jnp.float32)

        def chunk_body(i, carry):
            off = ebase + i * CH
            pltpu.sync_copy(src_hbm.at[pl.ds(off, CH)], src_c)
            pltpu.sync_copy(dst_hbm.at[pl.ds(off, CH)], dst_tab.at[pl.ds(i * CH, CH)])
            pltpu.sync_copy(ce1_hbm.at[pl.ds(off, CH)], ce1_c)
            pltpu.sync_copy(ce2_hbm.at[pl.ds(off, CH)], ce2_c)

            def grp(g, carry2):
                sl = pl.ds(g * L, L)
                si = src_c[sl]
                di = dst_tab[pl.ds(i * CH + g * L, L)]
                a_s = plsc.load_gather(a_tab, [si])
                a_d = plsc.load_gather(a_tab, [di])
                b_s = plsc.load_gather(b_tab, [si])
                b_d = plsc.load_gather(b_tab, [di])
                c1 = ce1_c[sl]
                c2 = ce2_c[sl]
                z1 = _lrelu(a_s + c1)
                z2 = _lrelu(a_d + c1)
                dz = z1 - z2
                p = jnp.exp(-jnp.abs(dz))
                q = 1.0 / (1.0 + p)
                al = jnp.where(dz >= 0.0, q, p * q)
                sc_ = _lrelu((1.0 + al) * b_s + (2.0 - al) * b_d + c2)
                alpha_c[sl] = al
                score_tab[pl.ds(i * CH + g * L, L)] = sc_
                rmax[...] = jnp.maximum(rmax[...], sc_)
                return carry2

            lax.fori_loop(0, CH // L, grp, 0)
            pltpu.sync_copy(alpha_c, alpha_out.at[pl.ds(off, CH)])
            return carry

        lax.fori_loop(0, EPT // CH, chunk_body, 0)

        pltpu.sync_copy(rmax, max_sh.at[s])
        plsc.subcore_barrier()
        pltpu.sync_copy(max_sh, maxloc)
        mv = maxloc[0, :]
        for k in range(1, NS):
            mv = jnp.maximum(mv, maxloc[k, :])
        mc = jnp.max(mv)
        mcv = jnp.broadcast_to(mc, (L,))

        @pl.when(s == 0)
        def _():
            rmax[...] = mcv
            pltpu.sync_copy(rmax, mc_out.at[c])

        def zr(g, carry):
            ssum_tab[pl.ds(g * L, L)] = jnp.zeros((L,), jnp.float32)
            return carry

        lax.fori_loop(0, NP // L, zr, 0)

        def chunk2(i, carry):
            def grp2(g, carry2):
                sl_t = pl.ds(i * CH + g * L, L)
                ev = jnp.exp(score_tab[sl_t] - mcv)
                e_c[pl.ds(g * L, L)] = ev
                di = dst_tab[sl_t]
                plsc.addupdate_scatter(ssum_tab, [di], ev)
                return carry2

            lax.fori_loop(0, CH // L, grp2, 0)
            pltpu.sync_copy(e_c, e_out.at[pl.ds(ebase + i * CH, CH)])
            return carry

        lax.fori_loop(0, EPT // CH, chunk2, 0)
        pltpu.sync_copy(ssum_tab, pssum_out.at[w])

    return body(a_pad, b_pad, src, dst, ce1, ce2)


def _sc_row_pass(x, src, dst, alpha0, e, mc, pssum):
    @functools.partial(
        pl.kernel,
        out_type=[
            jax.ShapeDtypeStruct((E, H), jnp.float32),
            jax.ShapeDtypeStruct((NC, NP, H), jnp.float32),
        ],
        mesh=_sc_mesh(),
        compiler_params=pltpu.CompilerParams(needs_layout_passes=False),
        scratch_types=[
            pltpu.VMEM((NP,), jnp.float32),
            pltpu.VMEM((NPT,), jnp.float32),
            pltpu.VMEM((NPT,), jnp.float32),
            pltpu.VMEM((NPT,), jnp.float32),
            pltpu.VMEM((NC, L), jnp.float32),
            pltpu.VMEM((CB,), jnp.int32),
            pltpu.VMEM((CB,), jnp.int32),
            pltpu.VMEM((CB,), jnp.float32),
            pltpu.VMEM((CB,), jnp.float32),
            pltpu.VMEM((RING, C2), jnp.int32),
            pltpu.VMEM((C, H), jnp.float32),
            pltpu.VMEM((C, H), jnp.float32),
            pltpu.VMEM((C, H), jnp.float32),
            pltpu.VMEM((SROWS, H), jnp.float32),
            pltpu.VMEM_SHARED((NP,), jnp.float32),
            pltpu.VMEM_SHARED((NP, H), jnp.float32),
            pltpu.SemaphoreType.DMA((RING,)),
            pltpu.SemaphoreType.DMA((RING,)),
            pltpu.SemaphoreType.DMA((RING,)),
            pltpu.SemaphoreType.DMA((RING,)),
            pltpu.SemaphoreType.DMA((RING,)),
        ],
    )
    def body(x_hbm, src_hbm, dst_hbm, al_hbm, e_hbm, mc_hbm, pssum_hbm,
             d_out, hpart_out,
             rinv_tab, row_buf, acc0_b, acc1_b, mcl, src_c, dst_c, al_c,
             bel_c, dst_ch, xs_b, xd_b, d_b, stage, rinv_sh, h_sh,
             lsem_xs, lsem_xd, lsem_dc, wsem_d, wsem_sc):
        c = lax.axis_index("c")
        s = lax.axis_index("s")
        w = s * NC + c
        ebase = w * EPT
        nb = s * NPT

        pltpu.sync_copy(mc_hbm, mcl)
        m0 = mcl[0, :]
        m1 = mcl[1, :]
        mg = jnp.maximum(m0, m1)
        sc0 = jnp.exp(m0 - mg)
        sc1 = jnp.exp(m1 - mg)
        cv = jnp.broadcast_to(c, (L,))
        myscale = jnp.where(cv == 0, sc0, sc1)

        def zacc(g, carry):
            sl = pl.ds(g * L, L)
            acc0_b[sl] = jnp.zeros((L,), jnp.float32)
            acc1_b[sl] = jnp.zeros((L,), jnp.float32)
            return carry

        lax.fori_loop(0, NPT // L, zacc, 0)
        for w2 in range(NW):
            pltpu.sync_copy(pssum_hbm.at[w2, pl.ds(nb, NPT)], row_buf)

            def accg(g, carry, _w2=w2):
                sl = pl.ds(g * L, L)
                if _w2 % NC == 0:
                    acc0_b[sl] = acc0_b[sl] + row_buf[sl]
                else:
                    acc1_b[sl] = acc1_b[sl] + row_buf[sl]
                return carry

            lax.fori_loop(0, NPT // L, accg, 0)

        def cg(g, carry):
            sl = pl.ds(g * L, L)
            tot = acc0_b[sl] * sc0 + acc1_b[sl] * sc1
            row_buf[sl] = 1.0 / jnp.maximum(tot, 1e-16)
            return carry

        lax.fori_loop(0, NPT // L, cg, 0)
        pltpu.sync_copy(row_buf, rinv_sh.at[pl.ds(nb, NPT)])

        def zs(r, carry):
            for j in range(H // L):
                stage[r, pl.ds(j * L, L)] = jnp.zeros((L,), jnp.float32)
            return carry

        lax.fori_loop(0, SROWS, zs, 0)
        for k in range(NPT // SROWS):
            pltpu.sync_copy(stage, h_sh.at[pl.ds(nb + k * SROWS, SROWS), :])

        plsc.subcore_barrier()
        pltpu.sync_copy(rinv_sh, rinv_tab)

        def _load_descs(boff, i):
            p = i % RING
            rows = pl.ds(p * C2, C2)
            off = boff + i * C2
            return [
                pltpu.make_async_copy(
                    x_hbm.at[src_c.at[pl.ds(i * C2, C2)]],
                    xs_b.at[rows, :], lsem_xs.at[p]),
                pltpu.make_async_copy(
                    x_hbm.at[dst_c.at[pl.ds(i * C2, C2)]],
                    xd_b.at[rows, :], lsem_xd.at[p]),
                pltpu.make_async_copy(
                    dst_hbm.at[pl.ds(off, C2)], dst_ch.at[p], lsem_dc.at[p]),
            ]

        def _start_writes(boff, i):
            p = i % RING
            rows = pl.ds(p * C2, C2)
            off = boff + i * C2
            pltpu.make_async_copy(
                d_b.at[rows, :], d_out.at[pl.ds(off, C2), :],
                wsem_d.at[p]).start()
            pltpu.async_copy(
                xs_b.at[rows, :], h_sh.at[dst_ch.at[p]], wsem_sc.at[p],
                add=True)

        def _wait_writes(boff, i):
            p = i % RING
            rows = pl.ds(p * C2, C2)
            off = boff + i * C2
            pltpu.make_async_copy(
                d_b.at[rows, :], d_out.at[pl.ds(off, C2), :],
                wsem_d.at[p]).wait()
            pltpu.make_async_copy(
                xs_b.at[rows, :], h_sh.at[dst_ch.at[p]],
                wsem_sc.at[p]).wait()

        def bat(ib, carry):
            boff = ebase + ib * CB
            pltpu.sync_copy(src_hbm.at[pl.ds(boff, CB)], src_c)
            pltpu.sync_copy(dst_hbm.at[pl.ds(boff, CB)], dst_c)
            pltpu.sync_copy(al_hbm.at[pl.ds(boff, CB)], al_c)
            pltpu.sync_copy(e_hbm.at[pl.ds(boff, CB)], bel_c)

            @plsc.parallel_loop(0, CB // L, 1, unroll=2)
            def _pg(g):
                sl = pl.ds(g * L, L)
                di = dst_c[sl]
                rv = plsc.load_gather(rinv_tab, [di])
                bel_c[sl] = bel_c[sl] * myscale * rv

            for ii in range(RING - 1):
                for d in _load_descs(boff, ii):
                    d.start()

            def half(i, carry2):
                p = i % RING
                for d in _load_descs(boff, i):
                    d.wait()

                @pl.when(i + RING - 1 < HB)
                def _():
                    @pl.when(i >= 1)
                    def _():
                        _wait_writes(boff, i - 1)

                    for d in _load_descs(boff, i + RING - 1):
                        d.start()

                @plsc.parallel_loop(0, C2, 1, unroll=8)
                def _rw(r):
                    rb = p * C2 + r
                    ridx = jnp.broadcast_to(i * C2 + r, (L,)).astype(jnp.int32)
                    alv = plsc.load_gather(al_c, [ridx])
                    blv = plsc.load_gather(bel_c, [ridx])
                    for j in range(H // L):
                        slj = pl.ds(j * L, L)
                        xsv = xs_b[rb, slj]
                        xdv = xd_b[rb, slj]
                        d_b[rb, slj] = alv * (xsv - xdv) + xdv
                        xs_b[rb, slj] = blv * xsv
                _start_writes(boff, i)
                return carry2

            lax.fori_loop(0, HB, half, 0)
            for ii in range(RING):
                _wait_writes(boff, HB - RING + ii)
            return carry

        lax.fori_loop(0, EPT // CB, bat, 0)
        plsc.subcore_barrier()

        for k in range(NPT // SROWS):
            rows = pl.ds(nb + k * SROWS, SROWS)
            pltpu.sync_copy(h_sh.at[rows, :], stage)
            pltpu.sync_copy(stage, hpart_out.at[c, rows, :])

    return body(x, src, dst, alpha0, e, mc, pssum)


def kernel(node_feat, edge_feat, guild_vec, edge_index, Wn, bn, We, be, Wg, bg, Wna, Wea):
    src = edge_index[0].astype(jnp.int32)
    dst = edge_index[1].astype(jnp.int32)
    wea = Wea[0]
    wna = Wna[0]

    Wab = jnp.stack([wea, wna], axis=1)
    WeAB = We.T @ Wab
    Wg2 = Wg.T @ Wab
    cg = (((be + bg) @ Wab))[None, :]
    cab = jnp.zeros((1, 2), jnp.float32)

    x, a_pad, b_pad = _node_proj(node_feat, Wn.T, bn[None, :], Wab, cab)
    ce1, ce2 = _ce_proj(edge_feat, guild_vec, WeAB, Wg2, cg)

    alpha0, e, mc, pssum = _sc_scalar_pass(a_pad, b_pad, src, dst, ce1, ce2)
    d, hpart = _sc_row_pass(x, src, dst, alpha0, e, mc, pssum)
    ef_new = _edge_final(edge_feat, We.T, be[None, :], d)
    h = _final_add(hpart, x)
    return h, ef_new
